# baseline (speedup 1.0000x reference)
import functools

import jax
import jax.numpy as jnp
from jax import lax
from jax.experimental import pallas as pl
from jax.experimental.pallas import tpu as pltpu
from jax.experimental.pallas import tpu_sc as plsc

BATCH = 4096
HIST = 200
EMBED_DIM = 64
DPAD = 128

_info = plsc.get_sparse_core_info()
_NC, _NS = _info.num_cores, _info.num_subcores
NW = _NC * _NS
LANES = BATCH // NW
NBUF = 3

_mesh = plsc.VectorSubcoreMesh(core_axis_name="c", subcore_axis_name="s")


@functools.partial(
    pl.kernel,
    mesh=_mesh,
    out_type=jax.ShapeDtypeStruct((HIST, EMBED_DIM, BATCH), jnp.float32),
    scratch_types=[
        pltpu.VMEM((HIST, LANES), jnp.int32),
        pltpu.VMEM((LANES, DPAD), jnp.float32),
        pltpu.VMEM((LANES, DPAD), jnp.float32),
        pltpu.VMEM((LANES, DPAD), jnp.float32),
        pltpu.VMEM((EMBED_DIM, LANES), jnp.float32),
        pltpu.VMEM((EMBED_DIM, LANES), jnp.float32),
        pltpu.VMEM((EMBED_DIM, LANES), jnp.float32),
        pltpu.SemaphoreType.DMA,
        pltpu.SemaphoreType.DMA,
        pltpu.SemaphoreType.DMA,
        pltpu.SemaphoreType.DMA,
        pltpu.SemaphoreType.DMA,
        pltpu.SemaphoreType.DMA,
    ],
    compiler_params=pltpu.CompilerParams(needs_layout_passes=False),
)
def _gather_kernel(idx_hbm, table_hbm, out_hbm, idx_all,
                   rows0, rows1, rows2, tr0, tr1, tr2,
                   sg0, sg1, sg2, sw0, sw1, sw2):
    wid = lax.axis_index("s") * _NC + lax.axis_index("c")
    lane0 = wid * LANES

    rows = (rows0, rows1, rows2)
    trv = (tr0, tr1, tr2)
    sg = (sg0, sg1, sg2)
    sw = (sw0, sw1, sw2)

    pltpu.sync_copy(idx_hbm.at[:, pl.ds(lane0, LANES)], idx_all)

    iotav = lax.iota(jnp.int32, 16)

    def gather_desc(h, b):
        src = table_hbm.at[idx_all.at[h]]
        return pltpu.make_async_copy(src, rows[b], sg[b])

    def wb_desc(h, b):
        dst = out_hbm.at[h, :, pl.ds(lane0, LANES)]
        return pltpu.make_async_copy(trv[b], dst, sw[b])

    ivs = tuple(iotav + (b0 * 16) for b0 in range(8))

    def transpose(b):
        rv, tv = rows[b], trv[b]

        def tb(d4, carry):
            for dd in range(4):
                jmod = jnp.bitwise_and(iotav + (4 * d4 + dd), 15)
                for c0 in range(0, EMBED_DIM, 16):
                    jv = jmod + c0
                    for b0 in range(8):
                        v = plsc.load_gather(rv, [ivs[b0], jv])
                        plsc.store_scatter(tv, [jv, ivs[b0]], v)
            return carry

        lax.fori_loop(0, 4, tb, 0)

    def body(h, b, wait_wb):
        if wait_wb:
            wb_desc(h - NBUF, b).wait()
        gather_desc(h, b).start()
        pb = (b - 1) % NBUF
        gather_desc(h - 1, pb).wait()
        transpose(pb)
        wb_desc(h - 1, pb).start()

    gather_desc(0, 0).start()
    body(1, 1, False)
    body(2, 2, False)

    def group(g, carry):
        for db in range(NBUF):
            h = NBUF * g + db
            body(h, db % NBUF, True)
        return carry

    lax.fori_loop(1, (HIST - 5) // NBUF + 1, group, 0)

    body(HIST - 2, (HIST - 2) % NBUF, True)
    body(HIST - 1, (HIST - 1) % NBUF, True)
    h = HIST - 1
    b = h % NBUF
    gather_desc(h, b).wait()
    transpose(b)
    wb_desc(h, b).start()
    for j in (HIST - 3, HIST - 2, HIST - 1):
        wb_desc(j, j % NBUF).wait()


def kernel(inputs, embedding):
    idx_t = jnp.transpose(inputs.astype(jnp.int32), (1, 0))
    table_pad = jnp.pad(embedding, ((0, 0), (0, DPAD - EMBED_DIM)))
    out_t = _gather_kernel(idx_t, table_pad)
    return jnp.transpose(out_t, (2, 0, 1))


# speedup vs baseline: 1.0224x; 1.0224x over previous
import functools

import jax
import jax.numpy as jnp
from jax import lax
from jax.experimental import pallas as pl
from jax.experimental.pallas import tpu as pltpu
from jax.experimental.pallas import tpu_sc as plsc

BATCH = 4096
HIST = 200
EMBED_DIM = 64
DPAD = 128

_info = plsc.get_sparse_core_info()
_NC, _NS = _info.num_cores, _info.num_subcores
NW = _NC * _NS
LANES = BATCH // NW
NBUF = 3

_mesh = plsc.VectorSubcoreMesh(core_axis_name="c", subcore_axis_name="s")


@functools.partial(
    pl.kernel,
    mesh=_mesh,
    out_type=jax.ShapeDtypeStruct((HIST, EMBED_DIM, BATCH), jnp.float32),
    scratch_types=[
        pltpu.VMEM((HIST, LANES), jnp.int32),
        pltpu.VMEM((LANES, DPAD), jnp.float32),
        pltpu.VMEM((LANES, DPAD), jnp.float32),
        pltpu.VMEM((LANES, DPAD), jnp.float32),
        pltpu.VMEM((EMBED_DIM, LANES), jnp.float32),
        pltpu.VMEM((EMBED_DIM, LANES), jnp.float32),
        pltpu.VMEM((EMBED_DIM, LANES), jnp.float32),
        pltpu.SemaphoreType.DMA,
        pltpu.SemaphoreType.DMA,
        pltpu.SemaphoreType.DMA,
        pltpu.SemaphoreType.DMA,
        pltpu.SemaphoreType.DMA,
        pltpu.SemaphoreType.DMA,
    ],
    compiler_params=pltpu.CompilerParams(needs_layout_passes=False),
)
def _gather_kernel(idx_hbm, table_hbm, out_hbm, idx_all,
                   rows0, rows1, rows2, tr0, tr1, tr2,
                   sg0, sg1, sg2, sw0, sw1, sw2):
    wid = lax.axis_index("s") * _NC + lax.axis_index("c")
    lane0 = wid * LANES

    rows = (rows0, rows1, rows2)
    trv = (tr0, tr1, tr2)
    sg = (sg0, sg1, sg2)
    sw = (sw0, sw1, sw2)

    pltpu.sync_copy(idx_hbm.at[:, pl.ds(lane0, LANES)], idx_all)

    iotav = lax.iota(jnp.int32, 16)

    def gather_desc(h, b):
        src = table_hbm.at[idx_all.at[h]]
        return pltpu.make_async_copy(src, rows[b], sg[b])

    def wb_desc(h, b):
        dst = out_hbm.at[h, :, pl.ds(lane0, LANES)]
        return pltpu.make_async_copy(trv[b], dst, sw[b])

    ivs = tuple(iotav + (b0 * 16) for b0 in range(8))

    def transpose(b):
        rv, tv = rows[b], trv[b]

        def tb(d, carry):
            jmod = jnp.bitwise_and(iotav + d, 15)
            for c0 in range(0, EMBED_DIM, 16):
                jv = jmod + c0
                for b0 in range(8):
                    v = plsc.load_gather(rv, [ivs[b0], jv])
                    plsc.store_scatter(tv, [jv, ivs[b0]], v)
            return carry

        lax.fori_loop(0, 16, tb, 0)

    def body(h, b, wait_wb):
        if wait_wb:
            wb_desc(h - NBUF, b).wait()
        gather_desc(h, b).start()
        pb = (b - 1) % NBUF
        gather_desc(h - 1, pb).wait()
        transpose(pb)
        wb_desc(h - 1, pb).start()

    gather_desc(0, 0).start()
    body(1, 1, False)
    body(2, 2, False)

    def group(g, carry):
        for db in range(NBUF):
            h = NBUF * g + db
            body(h, db % NBUF, True)
        return carry

    lax.fori_loop(1, (HIST - 5) // NBUF + 1, group, 0)

    body(HIST - 2, (HIST - 2) % NBUF, True)
    body(HIST - 1, (HIST - 1) % NBUF, True)
    h = HIST - 1
    b = h % NBUF
    gather_desc(h, b).wait()
    transpose(b)
    wb_desc(h, b).start()
    for j in (HIST - 3, HIST - 2, HIST - 1):
        wb_desc(j, j % NBUF).wait()


def kernel(inputs, embedding):
    idx_t = jnp.transpose(inputs.astype(jnp.int32), (1, 0))
    table_pad = jnp.pad(embedding, ((0, 0), (0, DPAD - EMBED_DIM)))
    out_t = _gather_kernel(idx_t, table_pad)
    return jnp.transpose(out_t, (2, 0, 1))


# speedup vs baseline: 1.0234x; 1.0010x over previous
import functools

import jax
import jax.numpy as jnp
from jax import lax
from jax.experimental import pallas as pl
from jax.experimental.pallas import tpu as pltpu
from jax.experimental.pallas import tpu_sc as plsc

BATCH = 4096
HIST = 200
EMBED_DIM = 64
DPAD = 128

_info = plsc.get_sparse_core_info()
_NC, _NS = _info.num_cores, _info.num_subcores
NW = _NC * _NS
LANES = BATCH // NW
NBUF = 3

_mesh = plsc.VectorSubcoreMesh(core_axis_name="c", subcore_axis_name="s")


@functools.partial(
    pl.kernel,
    mesh=_mesh,
    out_type=jax.ShapeDtypeStruct((HIST, EMBED_DIM, BATCH), jnp.float32),
    scratch_types=[
        pltpu.VMEM((HIST, LANES), jnp.int32),
        pltpu.VMEM((LANES, DPAD), jnp.float32),
        pltpu.VMEM((LANES, DPAD), jnp.float32),
        pltpu.VMEM((LANES, DPAD), jnp.float32),
        pltpu.VMEM((EMBED_DIM, LANES), jnp.float32),
        pltpu.VMEM((EMBED_DIM, LANES), jnp.float32),
        pltpu.VMEM((EMBED_DIM, LANES), jnp.float32),
        pltpu.SemaphoreType.DMA,
        pltpu.SemaphoreType.DMA,
        pltpu.SemaphoreType.DMA,
        pltpu.SemaphoreType.DMA,
        pltpu.SemaphoreType.DMA,
        pltpu.SemaphoreType.DMA,
    ],
    compiler_params=pltpu.CompilerParams(needs_layout_passes=False),
)
def _gather_kernel(idx_hbm, table_hbm, out_hbm, idx_all,
                   rows0, rows1, rows2, tr0, tr1, tr2,
                   sg0, sg1, sg2, sw0, sw1, sw2):
    wid = lax.axis_index("s") * _NC + lax.axis_index("c")
    lane0 = wid * LANES

    rows = (rows0, rows1, rows2)
    trv = (tr0, tr1, tr2)
    sg = (sg0, sg1, sg2)
    sw = (sw0, sw1, sw2)

    pltpu.sync_copy(idx_hbm.at[:, pl.ds(lane0, LANES)], idx_all)

    iotav = lax.iota(jnp.int32, 16)

    def gather_desc(h, b):
        src = table_hbm.at[idx_all.at[h]]
        return pltpu.make_async_copy(src, rows[b], sg[b])

    def wb_desc(h, b):
        dst = out_hbm.at[h, :, pl.ds(lane0, LANES)]
        return pltpu.make_async_copy(trv[b], dst, sw[b])

    ivs = tuple(iotav + (b0 * 16) for b0 in range(8))

    def transpose(b):
        rv, tv = rows[b], trv[b]

        def tb(d, carry):
            jmod = jnp.bitwise_and(iotav + d, 15)
            for c0 in range(0, EMBED_DIM, 16):
                jv = jmod + c0
                for b0 in range(8):
                    v = plsc.load_gather(rv, [ivs[b0], jv])
                    plsc.store_scatter(tv, [jv, ivs[b0]], v)
            return carry

        lax.fori_loop(0, 16, tb, 0)

    def body(h, b, wait_wb):
        if wait_wb:
            wb_desc(h - NBUF, b).wait()
        gather_desc(h, b).start()
        pb = (b - 1) % NBUF
        gather_desc(h - 1, pb).wait()
        transpose(pb)
        wb_desc(h - 1, pb).start()

    gather_desc(0, 0).start()
    body(1, 1, False)
    body(2, 2, False)

    def group(g, carry):
        for db in range(NBUF):
            h = NBUF * g + db
            body(h, db % NBUF, True)
        return carry

    lax.fori_loop(1, (HIST - 5) // NBUF + 1, group, 0)

    body(HIST - 2, (HIST - 2) % NBUF, True)
    body(HIST - 1, (HIST - 1) % NBUF, True)
    h = HIST - 1
    b = h % NBUF
    gather_desc(h, b).wait()
    transpose(b)
    wb_desc(h, b).start()
    for j in (HIST - 3, HIST - 2, HIST - 1):
        wb_desc(j, j % NBUF).wait()


def kernel(inputs, embedding):
    idx_t = jnp.transpose(inputs.astype(jnp.int32), (1, 0))
    table_pad = jnp.concatenate(
        [embedding,
         jnp.zeros((embedding.shape[0], DPAD - EMBED_DIM), embedding.dtype)],
        axis=1)
    out_t = _gather_kernel(idx_t, table_pad)
    return jnp.transpose(out_t, (2, 0, 1))
